# baseline (speedup 1.0000x reference)
import functools

import jax
import jax.numpy as jnp
from jax import lax
from jax.experimental import pallas as pl
from jax.experimental.pallas import tpu as pltpu
from jax.experimental.pallas import tpu_sc as plsc

NUM_CORES = 2
NUM_SUBCORES = 16
NUM_WORKERS = NUM_CORES * NUM_SUBCORES

BATCH = 16384
DIM = 64
PADDIM = 128

B_PER_W = BATCH // NUM_WORKERS


def _pair_body(i0, i1, t0, t1, o0, o1, idx_v, rows_v, gsem, ssem):
    wid = lax.axis_index("s") * NUM_CORES + lax.axis_index("c")
    base = wid * B_PER_W

    pltpu.sync_copy(i0.at[pl.ds(base, B_PER_W)], idx_v.at[0])
    pltpu.sync_copy(i1.at[pl.ds(base, B_PER_W)], idx_v.at[1])

    g0 = pltpu.async_copy(t0.at[idx_v.at[0]], rows_v.at[0], gsem.at[0])
    g1 = pltpu.async_copy(t1.at[idx_v.at[1]], rows_v.at[1], gsem.at[1])

    dst = pl.ds(base, B_PER_W), pl.ds(0, DIM)
    g0.wait()
    s0 = pltpu.async_copy(rows_v.at[0], o0.at[dst], ssem.at[0])
    g1.wait()
    s1 = pltpu.async_copy(rows_v.at[1], o1.at[dst], ssem.at[1])
    s0.wait()
    s1.wait()


@jax.jit
def _pair(i0, i1, t0, t1):
    mesh = plsc.VectorSubcoreMesh(core_axis_name="c", subcore_axis_name="s")
    out = jax.ShapeDtypeStruct((BATCH, PADDIM), jnp.float32)
    return pl.kernel(
        _pair_body,
        out_type=(out, out),
        mesh=mesh,
        compiler_params=pltpu.CompilerParams(use_tc_tiling_on_sc=False),
        scratch_types=[
            pltpu.VMEM((2, B_PER_W), jnp.int32),
            pltpu.VMEM((2, B_PER_W, DIM), jnp.float32),
            pltpu.SemaphoreType.DMA((2,)),
            pltpu.SemaphoreType.DMA((2,)),
        ],
    )(i0, i1, t0, t1)


def _single_body(i0, t0, o0, idx_v, rows_v, gsem, ssem):
    wid = lax.axis_index("s") * NUM_CORES + lax.axis_index("c")
    base = wid * B_PER_W
    pltpu.sync_copy(i0.at[pl.ds(base, B_PER_W)], idx_v)
    pltpu.async_copy(t0.at[idx_v], rows_v, gsem).wait()
    pltpu.async_copy(
        rows_v, o0.at[pl.ds(base, B_PER_W), pl.ds(0, DIM)], ssem).wait()


@jax.jit
def _single(i0, t0):
    mesh = plsc.VectorSubcoreMesh(core_axis_name="c", subcore_axis_name="s")
    return pl.kernel(
        _single_body,
        out_type=jax.ShapeDtypeStruct((BATCH, PADDIM), jnp.float32),
        mesh=mesh,
        compiler_params=pltpu.CompilerParams(use_tc_tiling_on_sc=False),
        scratch_types=[
            pltpu.VMEM((B_PER_W,), jnp.int32),
            pltpu.VMEM((B_PER_W, DIM), jnp.float32),
            pltpu.SemaphoreType.DMA,
            pltpu.SemaphoreType.DMA,
        ],
    )(i0, t0)


def kernel(head, relation, tail, timestamp,
           entity_table, relation_table, timestamp_table):
    rel_o, ts_o = _pair(relation.astype(jnp.int32),
                        timestamp.astype(jnp.int32),
                        relation_table, timestamp_table)
    head_o = _single(head.astype(jnp.int32), entity_table)
    tail_o = _single(tail.astype(jnp.int32), entity_table)
    return (head_o[:, :DIM], rel_o[:, :DIM],
            tail_o[:, :DIM], ts_o[:, :DIM])
